# Initial kernel scaffold; baseline (speedup 1.0000x reference)
import jax
import jax.numpy as jnp
from jax.experimental import pallas as pl


def kernel(input, coords, normalized):
    raise NotImplementedError("write your pallas kernel here")


# speedup vs baseline: 2.6249x; 2.6249x over previous
import functools

import jax
import jax.numpy as jnp
from jax import lax
from jax.experimental import pallas as pl
from jax.experimental.pallas import tpu as pltpu
from jax.experimental.pallas import tpu_sc as plsc

_L = 16


def _make_interp(nq, c, dims, chunk):
    d0, d1, d2 = dims
    nw = 32
    qpt = nq // nw
    nchunk = qpt // chunk
    nvec = chunk // _L
    mesh = plsc.VectorSubcoreMesh(core_axis_name="c", subcore_axis_name="s")

    @functools.partial(
        pl.kernel,
        mesh=mesh,
        out_type=jax.ShapeDtypeStruct((nq * c,), jnp.float32),
        compiler_params=pltpu.CompilerParams(
            use_tc_tiling_on_sc=False, needs_layout_passes=False),
        scratch_types=[
            pltpu.VMEM((chunk,), jnp.float32),
            pltpu.VMEM((chunk,), jnp.float32),
            pltpu.VMEM((chunk,), jnp.float32),
            pltpu.VMEM((8, chunk), jnp.int32),
            pltpu.VMEM((8 * chunk, c), jnp.float32),
            pltpu.VMEM((chunk * c,), jnp.float32),
            pltpu.SemaphoreType.DMA,
        ],
    )
    def interp(table, gx, gy, gz, out, cx, cy, cz, idx, rows, acc, sem):
        wid = lax.axis_index("s") * 2 + lax.axis_index("c")
        iota = lax.iota(jnp.int32, _L)
        sel8 = lax.shift_right_logical(iota, 3)
        chv = iota - sel8 * 8

        def chunk_body(t, _):
            qbase = wid * qpt + t * chunk
            pltpu.sync_copy(gx.at[pl.ds(qbase, chunk)], cx)
            pltpu.sync_copy(gy.at[pl.ds(qbase, chunk)], cy)
            pltpu.sync_copy(gz.at[pl.ds(qbase, chunk)], cz)

            frx, fry, frz = [], [], []
            for i in range(nvec):
                s = pl.ds(i * _L, _L)
                x, y, z = cx[s], cy[s], cz[s]
                xi = x.astype(jnp.int32)
                yi = y.astype(jnp.int32)
                zi = z.astype(jnp.int32)
                frx.append(x - xi.astype(jnp.float32))
                fry.append(y - yi.astype(jnp.float32))
                frz.append(z - zi.astype(jnp.float32))
                x0 = jnp.clip(xi, 0, d0 - 1) * (d1 * d2)
                x1 = jnp.clip(xi + 1, 0, d0 - 1) * (d1 * d2)
                y0 = jnp.clip(yi, 0, d1 - 1) * d2
                y1 = jnp.clip(yi + 1, 0, d1 - 1) * d2
                z0 = jnp.clip(zi, 0, d2 - 1)
                z1 = jnp.clip(zi + 1, 0, d2 - 1)
                for k, (px, py) in enumerate(
                        ((x0, y0), (x0, y1), (x1, y0), (x1, y1))):
                    pxy = px + py
                    idx[2 * k, s] = pxy + z0
                    idx[2 * k + 1, s] = pxy + z1

            copies = [
                pltpu.async_copy(table.at[idx.at[k]],
                                 rows.at[pl.ds(k * chunk, chunk)], sem)
                for k in range(8)
            ]
            for cp in copies:
                cp.wait()

            for i in range(nvec):
                for j in range(_L // 2):
                    eidx = sel8 + 2 * j
                    fx = frx[i].at[eidx].get(mode="promise_in_bounds")
                    fy = fry[i].at[eidx].get(mode="promise_in_bounds")
                    fz = frz[i].at[eidx].get(mode="promise_in_bounds")
                    off = (i * _L + 2 * j) * c
                    qv = sel8 + (2 * j + i * _L)
                    r = [plsc.load_gather(rows, [qv + k * chunk, chv])
                         for k in range(8)]
                    vz = [r[m] + fz * (r[m + 1] - r[m]) for m in (0, 2, 4, 6)]
                    vy = [vz[0] + fy * (vz[1] - vz[0]),
                          vz[2] + fy * (vz[3] - vz[2])]
                    acc[pl.ds(off, _L)] = vy[0] + fx * (vy[1] - vy[0])

            pltpu.sync_copy(acc, out.at[pl.ds(qbase * c, chunk * c)])

        lax.fori_loop(0, nchunk, chunk_body, None)

    return interp


def kernel(input, coords, normalized):
    c = input.shape[0]
    dims = input.shape[1:]
    nq = coords.shape[0]
    shape_f = jnp.asarray(dims, dtype=input.dtype)
    g = jnp.where(normalized != 0, (coords + 1.0) / 2.0 * (shape_f - 1.0),
                  coords)
    table = jnp.transpose(input, (1, 2, 3, 0)).reshape(-1, c)
    interp = _make_interp(nq, c, dims, chunk=128)
    out_flat = interp(table, g[:, 0], g[:, 1], g[:, 2])
    return out_flat.reshape(nq, c).T
